# baseline (speedup 1.0000x reference)
import functools

import jax
import jax.numpy as jnp
from jax import lax
from jax.experimental import pallas as pl
from jax.experimental.pallas import tpu as pltpu
from jax.experimental.pallas import tpu_sc as plsc

N = 10000
E = 160000
D = 256
NT = 16
NC = 2
NH = N // NC
E_PAD = 163840
EPT = E_PAD // NT
E_CAP = E_PAD + 4096
CHUNK = 32
GRP = NT * CHUNK
NPT = 312
NREM = NH - NT * NPT

_mesh = plsc.VectorSubcoreMesh(core_axis_name="c", subcore_axis_name="s")


def _bucket_body(row_hbm, col_hbm, ew_hbm,
                 brow_hbm, bcol_hbm, bew_hbm, cnt_hbm, degp_hbm,
                 row_v, col_v, ew_v, lrow_v, lcol_v, lew_v,
                 deg_v, tmp_v, cnts_v, drow_v, dcol_v, dew_v,
                 scounts, dsem):
    c = lax.axis_index("c")
    s = lax.axis_index("s")
    base = s * EPT
    lo = c * NH
    _i16 = jnp.zeros((16,), jnp.int32)
    _f16 = jnp.zeros((16,), jnp.float32)

    pltpu.sync_copy(row_hbm.at[pl.ds(base, EPT)], row_v)
    pltpu.sync_copy(col_hbm.at[pl.ds(base, EPT)], col_v)
    pltpu.sync_copy(ew_hbm.at[pl.ds(base, EPT)], ew_v)

    @pl.when(c == 0)
    def _zero_deg():
        def zbody(i, carry):
            deg_v[pl.ds(i * 16, 16)] = _f16
            return carry
        lax.fori_loop(0, N // 16, zbody, 0)

    lo_v = jnp.full((16,), 0, jnp.int32) + lo

    def scan_body(i, nacc):
        sl = pl.ds(i * 16, 16)
        cols = col_v[sl]
        rows = row_v[sl]
        ews = ew_v[sl]

        @pl.when(c == 0)
        def _deg_add():
            plsc.addupdate_scatter(deg_v, [cols], ews)

        m = jnp.logical_and(cols >= lo_v, cols < lo_v + NH)
        rank = plsc.cumsum(jnp.where(m, 1, 0))
        pos = nacc + rank - 1
        plsc.store_scatter(lrow_v, [pos], rows, mask=m)
        plsc.store_scatter(lcol_v, [pos], cols - lo_v, mask=m)
        plsc.store_scatter(lew_v, [pos], ews, mask=m)
        return nacc + plsc.all_reduce_population_count(m)

    nvec = lax.fori_loop(0, EPT // 16, scan_body, jnp.zeros((16,), jnp.int32))

    npad_vec = jnp.bitwise_and(nvec + 63, jnp.full((16,), ~63, jnp.int32))
    iota = lax.iota(jnp.int32, 16)
    for i in range(4):
        pos = nvec + i * 16 + iota
        plsc.store_scatter(lrow_v, [pos], _i16)
        plsc.store_scatter(lcol_v, [pos], _i16)
        plsc.store_scatter(lew_v, [pos], _f16)

    tmp_v[...] = npad_vec
    pltpu.sync_copy(tmp_v, scounts.at[pl.ds(s * 128, 16)])
    plsc.subcore_barrier()
    pltpu.sync_copy(scounts, cnts_v)

    offset = jnp.int32(0)
    total = jnp.int32(0)
    for sp in range(NT):
        n_sp = cnts_v[pl.ds(sp * 128, 16)][0]
        offset = offset + jnp.where(sp < s, n_sp, 0)
        total = total + n_sp

    offset = pl.multiple_of(offset, 64)
    total = pl.multiple_of(total, 64)

    my64 = npad_vec[0] // 64

    cbase = c * E_CAP

    def wbody(i, carry):
        dst = pl.ds(cbase + offset + i * 64, 64)
        src = pl.ds(i * 64, 64)
        pltpu.async_copy(lrow_v.at[src], brow_hbm.at[dst], dsem)
        pltpu.async_copy(lcol_v.at[src], bcol_hbm.at[dst], dsem)
        pltpu.async_copy(lew_v.at[src], bew_hbm.at[dst], dsem)
        return carry

    lax.fori_loop(0, my64, wbody, 0)

    tot_pad = ((total + (GRP - 1)) // GRP) * GRP
    gap64 = (tot_pad - total) // 64

    @pl.when(s == NT - 1)
    def _gap_fill():
        for i in range(4):
            drow_v[pl.ds(i * 16, 16)] = _i16
            dcol_v[pl.ds(i * 16, 16)] = _i16
            dew_v[pl.ds(i * 16, 16)] = _f16

        def gbody(i, carry):
            dst = pl.ds(cbase + total + i * 64, 64)
            pltpu.async_copy(drow_v, brow_hbm.at[dst], dsem)
            pltpu.async_copy(dcol_v, bcol_hbm.at[dst], dsem)
            pltpu.async_copy(dew_v, bew_hbm.at[dst], dsem)
            return carry

        lax.fori_loop(0, gap64, gbody, 0)

    @pl.when(s == 0)
    def _cnt_out():
        tmp_v[...] = jnp.full((16,), 0, jnp.int32) + tot_pad
        pltpu.sync_copy(tmp_v, cnt_hbm.at[pl.ds(c * 128, 16)])

    @pl.when(c == 0)
    def _deg_out():
        pltpu.sync_copy(deg_v, degp_hbm.at[s])

    def drainb(i, carry):
        dst = pl.ds(cbase, 64)
        pltpu.make_async_copy(lrow_v.at[pl.ds(0, 64)],
                              brow_hbm.at[dst], dsem).wait()
        pltpu.make_async_copy(lcol_v.at[pl.ds(0, 64)],
                              bcol_hbm.at[dst], dsem).wait()
        pltpu.make_async_copy(lew_v.at[pl.ds(0, 64)],
                              bew_hbm.at[dst], dsem).wait()
        return carry

    lax.fori_loop(0, my64, drainb, 0)

    @pl.when(s == NT - 1)
    def _gap_drain():
        lax.fori_loop(0, gap64, drainb, 0)


_bucket_call = functools.partial(
    pl.kernel,
    out_type=[
        jax.ShapeDtypeStruct((NC * E_CAP,), jnp.int32),
        jax.ShapeDtypeStruct((NC * E_CAP,), jnp.int32),
        jax.ShapeDtypeStruct((NC * E_CAP,), jnp.float32),
        jax.ShapeDtypeStruct((NC * 128,), jnp.int32),
        jax.ShapeDtypeStruct((NT, N), jnp.float32),
    ],
    mesh=_mesh,
    compiler_params=pltpu.CompilerParams(needs_layout_passes=False),
    scratch_types=[
        pltpu.VMEM((EPT,), jnp.int32),
        pltpu.VMEM((EPT,), jnp.int32),
        pltpu.VMEM((EPT,), jnp.float32),
        pltpu.VMEM((EPT + 64,), jnp.int32),
        pltpu.VMEM((EPT + 64,), jnp.int32),
        pltpu.VMEM((EPT + 64,), jnp.float32),
        pltpu.VMEM((N,), jnp.float32),
        pltpu.VMEM((16,), jnp.int32),
        pltpu.VMEM((NT * 128,), jnp.int32),
        pltpu.VMEM((64,), jnp.int32),
        pltpu.VMEM((64,), jnp.int32),
        pltpu.VMEM((64,), jnp.float32),
        pltpu.VMEM_SHARED((NT * 128,), jnp.int32),
        pltpu.SemaphoreType.DMA,
    ],
)(_bucket_body)


def _mm_body(degpt_ref, x_ref, w_ref, dis_ref, hp_ref, hsplit_ref):
    deg = jnp.sum(degpt_ref[...], axis=1, keepdims=True) + 1.0
    dis = lax.rsqrt(deg)
    dis_ref[...] = dis
    h = jnp.dot(x_ref[...], w_ref[...], preferred_element_type=jnp.float32)
    hp = h * dis
    hp_ref[...] = hp
    hsplit_ref[0] = hp[:, 0:128]
    hsplit_ref[1] = hp[:, 128:D]


_mm_call = pl.pallas_call(
    _mm_body,
    out_shape=[
        jax.ShapeDtypeStruct((N, 1), jnp.float32),
        jax.ShapeDtypeStruct((N, D), jnp.float32),
        jax.ShapeDtypeStruct((2, N, 128), jnp.float32),
    ],
    compiler_params=pltpu.CompilerParams(vmem_limit_bytes=100 * 1024 * 1024),
)


def _agg_body(hp_hbm, hsplit_hbm, brow_hbm, bcol_hbm, bew_hbm, cnt_hbm,
              agglo_hbm, agghi_hbm,
              idx_v, cidx_v, ew_v, cnt_v, gbuf, sbuf_lo, sbuf_hi,
              shared_lo, shared_hi,
              isem, csem, wsem, gsem, ssem, ssem2):
    c = lax.axis_index("c")
    s = lax.axis_index("s")
    node0 = c * NH

    pltpu.sync_copy(hsplit_hbm.at[0].at[pl.ds(node0 + s * NPT, NPT)],
                    shared_lo.at[pl.ds(s * NPT, NPT)])
    pltpu.sync_copy(hsplit_hbm.at[1].at[pl.ds(node0 + s * NPT, NPT)],
                    shared_hi.at[pl.ds(s * NPT, NPT)])

    @pl.when(s == 0)
    def _init_rem():
        pltpu.sync_copy(hsplit_hbm.at[0].at[pl.ds(node0 + NT * NPT, NREM)],
                        shared_lo.at[pl.ds(NT * NPT, NREM)])
        pltpu.sync_copy(hsplit_hbm.at[1].at[pl.ds(node0 + NT * NPT, NREM)],
                        shared_hi.at[pl.ds(NT * NPT, NREM)])

    pltpu.sync_copy(cnt_hbm.at[pl.ds(c * 128, 16)], cnt_v)
    trips = cnt_v[0:16][0] // GRP

    plsc.subcore_barrier()

    cbase = c * E_CAP

    def _fire(g, b):
        off = cbase + (g * NT + s) * CHUNK

        @pl.when(g >= 2)
        def _drain_prev():
            pltpu.make_async_copy(
                sbuf_lo.at[b], shared_lo.at[cidx_v.at[b]], ssem.at[b]).wait()
            pltpu.make_async_copy(
                sbuf_hi.at[b], shared_hi.at[cidx_v.at[b]], ssem2.at[b]).wait()

        pltpu.async_copy(brow_hbm.at[pl.ds(off, CHUNK)], idx_v.at[b],
                         isem.at[b])
        pltpu.async_copy(bcol_hbm.at[pl.ds(off, CHUNK)], cidx_v.at[b],
                         csem.at[b])
        pltpu.async_copy(bew_hbm.at[pl.ds(off, CHUNK)], ew_v.at[b],
                         wsem.at[b])
        pltpu.make_async_copy(brow_hbm.at[pl.ds(0, CHUNK)],
                              idx_v.at[b], isem.at[b]).wait()
        pltpu.async_copy(hp_hbm.at[idx_v.at[b]], gbuf.at[b], gsem.at[b])

    def _process(b):
        pltpu.make_async_copy(hp_hbm.at[idx_v.at[b]], gbuf.at[b],
                              gsem.at[b]).wait()
        pltpu.make_async_copy(bew_hbm.at[pl.ds(0, CHUNK)],
                              ew_v.at[b], wsem.at[b]).wait()

        def ebody(q, ecarry):
            wv = ew_v[b, pl.ds(q * 16, 16)]
            jb = q * 16
            for l in range(16):
                w = wv[l]
                for f in range(8):
                    sl = pl.ds(f * 16, 16)
                    sbuf_lo[b, jb + l, sl] = gbuf[b, jb + l, sl] * w
                for f in range(8):
                    sl = pl.ds(f * 16, 16)
                    sbuf_hi[b, jb + l, sl] = \
                        gbuf[b, jb + l, pl.ds(128 + f * 16, 16)] * w
            return ecarry

        lax.fori_loop(0, CHUNK // 16, ebody, 0)
        pltpu.make_async_copy(bcol_hbm.at[pl.ds(0, CHUNK)],
                              cidx_v.at[b], csem.at[b]).wait()
        pltpu.async_copy(sbuf_lo.at[b], shared_lo.at[cidx_v.at[b]],
                         ssem.at[b], add=True)
        pltpu.async_copy(sbuf_hi.at[b], shared_hi.at[cidx_v.at[b]],
                         ssem2.at[b], add=True)

    def it_body(g, carry):
        p = lax.rem(g, 2)

        @pl.when(jnp.logical_and(p == 0, g < trips))
        def _a0():
            _fire(g, 0)

        @pl.when(jnp.logical_and(p == 1, g < trips))
        def _a1():
            _fire(g, 1)

        @pl.when(jnp.logical_and(p == 1, g > 0))
        def _b0():
            _process(0)

        @pl.when(jnp.logical_and(p == 0, g > 0))
        def _b1():
            _process(1)

        return carry

    lax.fori_loop(0, trips + 1, it_body, 0)

    for b in range(2):
        @pl.when(trips > b)
        def _final_drain(b=b):
            pltpu.make_async_copy(
                sbuf_lo.at[b], shared_lo.at[cidx_v.at[b]], ssem.at[b]).wait()
            pltpu.make_async_copy(
                sbuf_hi.at[b], shared_hi.at[cidx_v.at[b]], ssem2.at[b]).wait()

    plsc.subcore_barrier()

    pltpu.sync_copy(shared_lo.at[pl.ds(s * NPT, NPT)],
                    agglo_hbm.at[pl.ds(node0 + s * NPT, NPT)])
    pltpu.sync_copy(shared_hi.at[pl.ds(s * NPT, NPT)],
                    agghi_hbm.at[pl.ds(node0 + s * NPT, NPT)])

    @pl.when(s == 0)
    def _out_rem():
        pltpu.sync_copy(shared_lo.at[pl.ds(NT * NPT, NREM)],
                        agglo_hbm.at[pl.ds(node0 + NT * NPT, NREM)])
        pltpu.sync_copy(shared_hi.at[pl.ds(NT * NPT, NREM)],
                        agghi_hbm.at[pl.ds(node0 + NT * NPT, NREM)])


_agg_call = functools.partial(
    pl.kernel,
    out_type=[
        jax.ShapeDtypeStruct((N, 128), jnp.float32),
        jax.ShapeDtypeStruct((N, 128), jnp.float32),
    ],
    mesh=_mesh,
    compiler_params=pltpu.CompilerParams(needs_layout_passes=False),
    scratch_types=[
        pltpu.VMEM((2, CHUNK), jnp.int32),
        pltpu.VMEM((2, CHUNK), jnp.int32),
        pltpu.VMEM((2, CHUNK), jnp.float32),
        pltpu.VMEM((16,), jnp.int32),
        pltpu.VMEM((2, CHUNK, D), jnp.float32),
        pltpu.VMEM((2, CHUNK, 128), jnp.float32),
        pltpu.VMEM((2, CHUNK, 128), jnp.float32),
        pltpu.VMEM_SHARED((NH, 128), jnp.float32),
        pltpu.VMEM_SHARED((NH, 128), jnp.float32),
        pltpu.SemaphoreType.DMA((2,)),
        pltpu.SemaphoreType.DMA((2,)),
        pltpu.SemaphoreType.DMA((2,)),
        pltpu.SemaphoreType.DMA((2,)),
        pltpu.SemaphoreType.DMA((2,)),
        pltpu.SemaphoreType.DMA((2,)),
    ],
)(_agg_body)


def _bn_body(agglo_ref, agghi_ref, dis_ref, x_ref, b_ref, gamma_ref,
             beta_ref, out_ref):
    agg = jnp.concatenate([agglo_ref[...], agghi_ref[...]], axis=1)
    y = agg * dis_ref[...] + b_ref[...]
    mean = jnp.mean(y, axis=0, keepdims=True)
    yc = y - mean
    var = jnp.mean(yc * yc, axis=0, keepdims=True)
    out_ref[...] = yc * lax.rsqrt(var + 1e-5) * gamma_ref[...] \
        + beta_ref[...] + x_ref[...]


_bn_call = pl.pallas_call(
    _bn_body,
    out_shape=jax.ShapeDtypeStruct((N, D), jnp.float32),
    compiler_params=pltpu.CompilerParams(vmem_limit_bytes=100 * 1024 * 1024),
)


def kernel(x, edge_index, edge_weight, W, b, gamma, beta):
    row = edge_index[0].astype(jnp.int32)
    col = edge_index[1].astype(jnp.int32)
    ew = edge_weight.astype(jnp.float32)
    pad = E_PAD - E
    row_p = jnp.concatenate([row, jnp.zeros((pad,), jnp.int32)])
    col_p = jnp.concatenate([col, jnp.zeros((pad,), jnp.int32)])
    ew_p = jnp.concatenate([ew, jnp.zeros((pad,), jnp.float32)])

    brow, bcol, bew, cnt, degp = _bucket_call(row_p, col_p, ew_p)
    degpt = degp.T
    dis, hp, hsplit = _mm_call(degpt, x, W)
    agglo, agghi = _agg_call(hp, hsplit, brow, bcol, bew, cnt)
    out = _bn_call(agglo, agghi, dis, x,
                   b.reshape(1, D), gamma.reshape(1, D), beta.reshape(1, D))
    return out


# speedup vs baseline: 1.9766x; 1.9766x over previous
import functools

import jax
import jax.numpy as jnp
from jax import lax
from jax.experimental import pallas as pl
from jax.experimental.pallas import tpu as pltpu
from jax.experimental.pallas import tpu_sc as plsc

N = 10000
E = 160000
D = 256
DH = 128
NT = 16
NC = 2
E_PAD = 163840
EPT_DEG = E_PAD // (NC * NT)
EPT_AGG = E_PAD // NT
CHUNK = 80
NPT = 624
NREM = N - NT * NPT

_mesh = plsc.VectorSubcoreMesh(core_axis_name="c", subcore_axis_name="s")


def _deg_body(col_hbm, ew_hbm, out_hbm, deg_v, col_v, ew_v):
    c = lax.axis_index("c")
    s = lax.axis_index("s")
    wid = s * NC + c
    base = wid * EPT_DEG
    pltpu.sync_copy(col_hbm.at[pl.ds(base, EPT_DEG)], col_v)
    pltpu.sync_copy(ew_hbm.at[pl.ds(base, EPT_DEG)], ew_v)

    zeros = jnp.zeros((16,), jnp.float32)

    def zbody(i, carry):
        deg_v[pl.ds(i * 16, 16)] = zeros
        return carry

    lax.fori_loop(0, N // 16, zbody, 0)

    def ebody(i, carry):
        idx = col_v[pl.ds(i * 16, 16)]
        w = ew_v[pl.ds(i * 16, 16)]
        plsc.addupdate_scatter(deg_v, [idx], w)
        return carry

    lax.fori_loop(0, EPT_DEG // 16, ebody, 0)
    pltpu.sync_copy(deg_v, out_hbm.at[wid])


_deg_call = functools.partial(
    pl.kernel,
    out_type=jax.ShapeDtypeStruct((NC * NT, N), jnp.float32),
    mesh=_mesh,
    compiler_params=pltpu.CompilerParams(needs_layout_passes=False),
    scratch_types=[
        pltpu.VMEM((N,), jnp.float32),
        pltpu.VMEM((EPT_DEG,), jnp.int32),
        pltpu.VMEM((EPT_DEG,), jnp.float32),
    ],
)(_deg_body)


def _mm_body(degpt_ref, x_ref, w_ref, dis_ref, hcat_ref):
    deg = jnp.sum(degpt_ref[...], axis=1, keepdims=True) + 1.0
    dis = lax.rsqrt(deg)
    dis_ref[...] = dis
    h = jnp.dot(x_ref[...], w_ref[...], preferred_element_type=jnp.float32)
    hp = h * dis
    hcat_ref[0] = hp[:, 0:DH]
    hcat_ref[1] = hp[:, DH:D]


_mm_call = pl.pallas_call(
    _mm_body,
    out_shape=[
        jax.ShapeDtypeStruct((N, 1), jnp.float32),
        jax.ShapeDtypeStruct((2, N, DH), jnp.float32),
    ],
    compiler_params=pltpu.CompilerParams(vmem_limit_bytes=100 * 1024 * 1024),
)


K = 2
NBUF = 2 * K
NIT = EPT_AGG // (K * CHUNK)


def _agg_body(hcat_hbm, row_hbm, col_hbm, ew_hbm, agg_hbm,
              idx_v, cidx_v, ew_v, gbuf, shared_agg,
              isem, csem, wsem, gsem, ssem):
    c = lax.axis_index("c")
    s = lax.axis_index("s")
    base = s * EPT_AGG
    htab = hcat_hbm.at[c]
    atab = agg_hbm.at[c]

    pltpu.sync_copy(htab.at[pl.ds(s * NPT, NPT)],
                    shared_agg.at[pl.ds(s * NPT, NPT)])

    @pl.when(s == 0)
    def _init_rem():
        pltpu.sync_copy(htab.at[pl.ds(NT * NPT, NREM)],
                        shared_agg.at[pl.ds(NT * NPT, NREM)])

    plsc.subcore_barrier()

    def _fire(g, ps):
        poff = base + g * (K * CHUNK)
        for j in range(K):
            b = ps * K + j
            off = poff + j * CHUNK

            @pl.when(g >= 2)
            def _drain_prev(b=b):
                pltpu.make_async_copy(
                    gbuf.at[b], shared_agg.at[cidx_v.at[b]], ssem.at[b]
                ).wait()

            pltpu.async_copy(row_hbm.at[pl.ds(off, CHUNK)], idx_v.at[b],
                             isem.at[b])
            pltpu.async_copy(col_hbm.at[pl.ds(off, CHUNK)], cidx_v.at[b],
                             csem.at[b])
            pltpu.async_copy(ew_hbm.at[pl.ds(off, CHUNK)], ew_v.at[b],
                             wsem.at[b])
        for j in range(K):
            b = ps * K + j
            pltpu.make_async_copy(row_hbm.at[pl.ds(base, CHUNK)],
                                  idx_v.at[b], isem.at[b]).wait()
            pltpu.async_copy(htab.at[idx_v.at[b]], gbuf.at[b], gsem.at[b])

    def _process(g, ps):
        poff = base + g * (K * CHUNK)
        for j in range(K):
            b = ps * K + j
            off = poff + j * CHUNK
            pltpu.make_async_copy(htab.at[idx_v.at[b]], gbuf.at[b],
                                  gsem.at[b]).wait()
            pltpu.make_async_copy(ew_hbm.at[pl.ds(base, CHUNK)],
                                  ew_v.at[b], wsem.at[b]).wait()

            def ebody(q, ecarry, b=b):
                wv = ew_v[b, pl.ds(q * 16, 16)]
                jb = q * 16
                for l in range(16):
                    w = wv[l]
                    for f in range(DH // 16):
                        sl = pl.ds(f * 16, 16)
                        gbuf[b, jb + l, sl] = gbuf[b, jb + l, sl] * w
                return ecarry

            lax.fori_loop(0, CHUNK // 16, ebody, 0)
            pltpu.make_async_copy(col_hbm.at[pl.ds(base, CHUNK)],
                                  cidx_v.at[b], csem.at[b]).wait()
            pltpu.async_copy(gbuf.at[b], shared_agg.at[cidx_v.at[b]],
                             ssem.at[b], add=True)

    def it_body(g, carry):
        p = lax.rem(g, 2)

        @pl.when(jnp.logical_and(p == 0, g < NIT))
        def _a0():
            _fire(g, 0)

        @pl.when(jnp.logical_and(p == 1, g < NIT))
        def _a1():
            _fire(g, 1)

        @pl.when(jnp.logical_and(p == 1, g > 0))
        def _b0():
            _process(g - 1, 0)

        @pl.when(jnp.logical_and(p == 0, g > 0))
        def _b1():
            _process(g - 1, 1)

        return carry

    lax.fori_loop(0, NIT + 1, it_body, 0)

    for b in range(NBUF):
        pltpu.make_async_copy(
            gbuf.at[b], shared_agg.at[cidx_v.at[b]], ssem.at[b]).wait()
    plsc.subcore_barrier()

    pltpu.sync_copy(shared_agg.at[pl.ds(s * NPT, NPT)],
                    atab.at[pl.ds(s * NPT, NPT)])

    @pl.when(s == 0)
    def _out_rem():
        pltpu.sync_copy(shared_agg.at[pl.ds(NT * NPT, NREM)],
                        atab.at[pl.ds(NT * NPT, NREM)])


_agg_call = functools.partial(
    pl.kernel,
    out_type=jax.ShapeDtypeStruct((NC, N, DH), jnp.float32),
    mesh=_mesh,
    compiler_params=pltpu.CompilerParams(needs_layout_passes=False),
    scratch_types=[
        pltpu.VMEM((NBUF, CHUNK), jnp.int32),
        pltpu.VMEM((NBUF, CHUNK), jnp.int32),
        pltpu.VMEM((NBUF, CHUNK), jnp.float32),
        pltpu.VMEM((NBUF, CHUNK, DH), jnp.float32),
        pltpu.VMEM_SHARED((N, DH), jnp.float32),
        pltpu.SemaphoreType.DMA((NBUF,)),
        pltpu.SemaphoreType.DMA((NBUF,)),
        pltpu.SemaphoreType.DMA((NBUF,)),
        pltpu.SemaphoreType.DMA((NBUF,)),
        pltpu.SemaphoreType.DMA((NBUF,)),
    ],
)(_agg_body)


def _bn_body(agg_ref, dis_ref, x_ref, b_ref, gamma_ref, beta_ref, out_ref):
    agg = jnp.concatenate([agg_ref[0:N, :], agg_ref[N:2 * N, :]], axis=1)
    y = agg * dis_ref[...] + b_ref[...]
    mean = jnp.mean(y, axis=0, keepdims=True)
    yc = y - mean
    var = jnp.mean(yc * yc, axis=0, keepdims=True)
    out_ref[...] = yc * lax.rsqrt(var + 1e-5) * gamma_ref[...] \
        + beta_ref[...] + x_ref[...]


_bn_call = pl.pallas_call(
    _bn_body,
    out_shape=jax.ShapeDtypeStruct((N, D), jnp.float32),
    compiler_params=pltpu.CompilerParams(vmem_limit_bytes=100 * 1024 * 1024),
)


def kernel(x, edge_index, edge_weight, W, b, gamma, beta):
    row = edge_index[0].astype(jnp.int32)
    col = edge_index[1].astype(jnp.int32)
    ew = edge_weight.astype(jnp.float32)
    pad = E_PAD - E
    row_p = jnp.concatenate([row, jnp.zeros((pad,), jnp.int32)])
    col_p = jnp.concatenate([col, jnp.zeros((pad,), jnp.int32)])
    ew_p = jnp.concatenate([ew, jnp.zeros((pad,), jnp.float32)])

    degp = _deg_call(col_p, ew_p)
    degpt = degp.T
    dis, hcat = _mm_call(degpt, x, W)
    agg = _agg_call(hcat, row_p, col_p, ew_p)
    agg = agg.reshape(NC * N, DH)
    out = _bn_call(agg, dis, x,
                   b.reshape(1, D), gamma.reshape(1, D), beta.reshape(1, D))
    return out
